# baseline (speedup 1.0000x reference)
import functools

import jax
import jax.numpy as jnp
from jax.experimental import pallas as pl

K = 8192
D = 256
TN = 1024


TK = 2048


def _vq_kernel(z_ref, e_ref, zn_ref, en_ref, out_ref):
    z = z_ref[...]
    zn = zn_ref[...]
    e = e_ref[...]
    en = en_ref[...]
    scores = jax.lax.dot_general(
        -2.0 * z, e, (((1,), (1,)), ((), ())), preferred_element_type=jnp.float32
    )

    SR = 64
    nsl = K // 128
    bvs = []
    bjs = []
    for rg in range(TN // SR):
        znr = zn[rg * SR:(rg + 1) * SR, :]
        best_v = jnp.full((SR, 128), jnp.inf, dtype=jnp.float32)
        best_j = jnp.zeros((SR, 128), dtype=jnp.int32)
        for j in range(nsl):
            d = (znr + en[:, j * 128:(j + 1) * 128]) \
                + scores[rg * SR:(rg + 1) * SR, j * 128:(j + 1) * 128]
            lt = d < best_v
            best_v = jnp.where(lt, d, best_v)
            best_j = jnp.where(lt, j, best_j)
        bvs.append(best_v)
        bjs.append(best_j)

    bv = jnp.concatenate(bvs, axis=0)
    bj = jnp.concatenate(bjs, axis=0)
    lane = jax.lax.broadcasted_iota(jnp.int32, (TN, 128), 1)
    gidx = bj * 128 + lane
    m = jnp.min(bv, axis=1, keepdims=True)
    idx = jnp.min(jnp.where(bv == m, gidx, K), axis=1)
    out_ref[...] = idx.astype(jnp.int32)


def kernel(z_e_x, emb_weight):
    B, Dm, H, W = z_e_x.shape
    N = B * H * W
    z_r = jnp.transpose(z_e_x, (0, 2, 3, 1)).reshape(N, Dm)
    z_norm = (z_r ** 2).sum(axis=1, keepdims=True)
    e_norm = (emb_weight ** 2).sum(axis=1, keepdims=True).T
    latents = pl.pallas_call(
        _vq_kernel,
        grid=(N // TN,),
        in_specs=[
            pl.BlockSpec((TN, D), lambda i: (i, 0)),
            pl.BlockSpec((K, D), lambda i: (0, 0)),
            pl.BlockSpec((TN, 1), lambda i: (i, 0)),
            pl.BlockSpec((1, K), lambda i: (0, 0)),
        ],
        out_specs=pl.BlockSpec((TN,), lambda i: (i,)),
        out_shape=jax.ShapeDtypeStruct((N,), jnp.int32),
    )(z_r, emb_weight, z_norm, e_norm)

    return latents.reshape(B, H, W)


# speedup vs baseline: 1.0294x; 1.0294x over previous
import functools

import jax
import jax.numpy as jnp
from jax.experimental import pallas as pl

K = 8192
D = 256
TN = 1024


TK = 2048


def _vq_kernel(z_ref, e_ref, zn_ref, en_ref, out_ref):
    z = z_ref[...]
    zn = zn_ref[...]
    e = e_ref[...]
    en = en_ref[...]
    scores = jax.lax.dot_general(
        -2.0 * z, e, (((1,), (1,)), ((), ())), preferred_element_type=jnp.float32
    )

    SR = 64
    nsl = K // 128
    lane = jax.lax.broadcasted_iota(jnp.int32, (SR, 128), 1)
    for rg in range(TN // SR):
        znr = zn[rg * SR:(rg + 1) * SR, :]
        best_v = jnp.full((SR, 128), jnp.inf, dtype=jnp.float32)
        best_j = jnp.zeros((SR, 128), dtype=jnp.int32)
        for j in range(nsl):
            d = (znr + en[:, j * 128:(j + 1) * 128]) \
                + scores[rg * SR:(rg + 1) * SR, j * 128:(j + 1) * 128]
            lt = d < best_v
            best_v = jnp.where(lt, d, best_v)
            best_j = jnp.where(lt, j, best_j)
        gidx = best_j * 128 + lane
        m = jnp.min(best_v, axis=1, keepdims=True)
        idx = jnp.min(jnp.where(best_v == m, gidx, K), axis=1)
        out_ref[rg * SR:(rg + 1) * SR] = idx.astype(jnp.int32)


def kernel(z_e_x, emb_weight):
    B, Dm, H, W = z_e_x.shape
    N = B * H * W
    z_r = jnp.transpose(z_e_x, (0, 2, 3, 1)).reshape(N, Dm)
    z_norm = (z_r ** 2).sum(axis=1, keepdims=True)
    e_norm = (emb_weight ** 2).sum(axis=1, keepdims=True).T
    latents = pl.pallas_call(
        _vq_kernel,
        grid=(N // TN,),
        in_specs=[
            pl.BlockSpec((TN, D), lambda i: (i, 0)),
            pl.BlockSpec((K, D), lambda i: (0, 0)),
            pl.BlockSpec((TN, 1), lambda i: (i, 0)),
            pl.BlockSpec((1, K), lambda i: (0, 0)),
        ],
        out_specs=pl.BlockSpec((TN,), lambda i: (i,)),
        out_shape=jax.ShapeDtypeStruct((N,), jnp.int32),
    )(z_r, emb_weight, z_norm, e_norm)

    return latents.reshape(B, H, W)
